# Initial kernel scaffold; baseline (speedup 1.0000x reference)
import jax
import jax.numpy as jnp
from jax.experimental import pallas as pl


def kernel(x, edge_index, Wn, bn, g1, b1, Wk, bk, g2, b2, We, be, Wp, bp, Wih, bih, Whh, bhh, g_ln, b_ln, Wc, bc, g3, b3):
    raise NotImplementedError("write your pallas kernel here")


# speedup vs baseline: 3.4154x; 3.4154x over previous
import functools

import jax
import jax.numpy as jnp
from jax import lax
from jax.experimental import pallas as pl
from jax.experimental.pallas import tpu as pltpu
from jax.experimental.pallas import tpu_sc as plsc

F32 = jnp.float32

_NC = 2
_NS = 16
_NW = _NC * _NS
_C = 80


def _node_pre_body(x_ref, wn_ref, bn_ref, g1_ref, b1_ref, wp_ref, bp_ref,
                   wl_ref, bl_ref, np_ref, hv_ref, l2_ref):
    x = x_ref[...]
    t = jnp.dot(x, wn_ref[...], preferred_element_type=F32) + bn_ref[...]
    mask = lax.broadcasted_iota(jnp.int32, t.shape, 1) < 20
    m = jnp.sum(t, axis=1, keepdims=True) * (1.0 / 20.0)
    d = jnp.where(mask, t - m, 0.0)
    v = jnp.sum(d * d, axis=1, keepdims=True) * (1.0 / 20.0)
    y = d * lax.rsqrt(v + 1e-5) * g1_ref[...] + b1_ref[...]
    np_ref[...] = jnp.where(mask, jnp.maximum(y, 0.0), 0.0)
    hv_ref[...] = jnp.dot(x, wp_ref[...], preferred_element_type=F32) + bp_ref[...]
    l2_ref[...] = jnp.dot(x, wl_ref[...], preferred_element_type=F32) + bl_ref[...]


def _kron_body(sp_ref, dp_ref, wk_ref, bk_ref, g2_ref, b2_ref, ek_ref):
    sp = sp_ref[...]
    dp = dp_ref[...]
    kron = jnp.concatenate([sp[:, a:a + 1] * dp for a in range(20)], axis=1)
    ek = jnp.dot(kron, wk_ref[...], preferred_element_type=F32) + bk_ref[...]
    m = jnp.mean(ek, axis=1, keepdims=True)
    d = ek - m
    v = jnp.mean(d * d, axis=1, keepdims=True)
    y = d * lax.rsqrt(v + 1e-5) * g2_ref[...] + b2_ref[...]
    ek_ref[...] = jnp.maximum(y, 0.0)


def _final_body(x_ref, u0_ref, u1_ref, d0_ref, d1_ref, k0_ref, k1_ref,
                wih_ref, bih_ref, whh_ref, bhh_ref, gl_ref, bl_ref,
                wc1_ref, wc2_ref, bc_ref, g3_ref, b3_ref, out_ref):
    x = x_ref[...]
    den = d0_ref[:, :1] + d1_ref[:, :1]
    ctx = jnp.maximum((u0_ref[...] + u1_ref[...]) / (den + 1e-9), 0.0)
    gi = jnp.dot(ctx, wih_ref[...], preferred_element_type=F32) + bih_ref[...]
    gh = jnp.dot(x, whh_ref[...], preferred_element_type=F32) + bhh_ref[...]
    r = jax.nn.sigmoid(gi[:, :128] + gh[:, :128])
    z = jax.nn.sigmoid(gi[:, 128:256] + gh[:, 128:256])
    nc = jnp.tanh(gi[:, 256:] + r * gh[:, 256:])
    h = (1.0 - z) * nc + z * x
    g = jnp.maximum(h, 0.0)
    m = jnp.mean(g, axis=1, keepdims=True)
    d = g - m
    v = jnp.mean(d * d, axis=1, keepdims=True)
    g = d * lax.rsqrt(v + 1e-5) * gl_ref[...] + bl_ref[...]
    kf = k0_ref[...] + k1_ref[...]
    o = (jnp.dot(g, wc1_ref[...], preferred_element_type=F32)
         + jnp.dot(kf, wc2_ref[...], preferred_element_type=F32) + bc_ref[...])
    m2 = jnp.mean(o, axis=1, keepdims=True)
    d2 = o - m2
    v2 = jnp.mean(d2 * d2, axis=1, keepdims=True)
    y = d2 * lax.rsqrt(v2 + 1e-5) * g3_ref[...] + b3_ref[...]
    out_ref[...] = jnp.maximum(y, 0.0)


def _edge_pass_body(src_h, dst_h, np_h, hv_h, ld_h, ls_h, zu_h, zd_h,
                    u0_out, u1_out, d0_out, d1_out, sp_out, dp_out,
                    sidx, didx, spb, dpb, gbuf, wrow, ldv, lsv,
                    uacc, dacc, sem1, sem2, sem3,
                    *, n_pad, epw, nchunk):
    c = lax.axis_index("c")
    s = lax.axis_index("s")
    wid = s * _NC + c
    rz = n_pad // _NS
    pltpu.sync_copy(zu_h, uacc.at[pl.ds(s * rz, rz)])
    pltpu.sync_copy(zd_h, dacc.at[pl.ds(s * rz, rz)])
    pltpu.sync_copy(ld_h, ldv)
    pltpu.sync_copy(ls_h, lsv)
    for r in range(_C):
        wrow[r] = jnp.zeros((16,), F32)
    plsc.subcore_barrier()

    iota16 = lax.iota(jnp.int32, 16)
    zeros16 = jnp.zeros((16,), jnp.int32)

    def chunk(i, carry):
        base = wid * epw + i * _C
        pltpu.sync_copy(src_h.at[pl.ds(base, _C)], sidx)
        pltpu.sync_copy(dst_h.at[pl.ds(base, _C)], didx)
        cp1 = pltpu.async_copy(hv_h.at[sidx], gbuf, sem1)
        cp2 = pltpu.async_copy(np_h.at[sidx], spb, sem2)
        cp3 = pltpu.async_copy(np_h.at[didx], dpb, sem3)
        cp2.wait()
        pltpu.sync_copy(spb, sp_out.at[pl.ds(base, _C)])
        cp3.wait()
        pltpu.sync_copy(dpb, dp_out.at[pl.ds(base, _C)])
        for j in range(_C // 16):
            sv = sidx[pl.ds(j * 16, 16)]
            dv = didx[pl.ds(j * 16, 16)]
            lv = plsc.load_gather(ldv, [dv]) + plsc.load_gather(lsv, [sv])
            w = jnp.exp(jnp.maximum(lv, 0.0))
            plsc.store_scatter(wrow, [iota16 + (j * 16), zeros16], w)
        cp1.wait()

        def scale(cc, carry2):
            wv = plsc.load_gather(wrow, [jnp.full((16,), 0, jnp.int32) + cc,
                                         zeros16])
            for r in range(8):
                gbuf[cc, pl.ds(r * 16, 16)] = gbuf[cc, pl.ds(r * 16, 16)] * wv
            return carry2

        lax.fori_loop(0, _C, scale, 0)
        pltpu.sync_copy(gbuf, uacc.at[didx], add=True)
        pltpu.sync_copy(wrow, dacc.at[didx], add=True)
        return carry

    lax.fori_loop(0, nchunk, chunk, 0)
    plsc.subcore_barrier()
    rows = pl.ds(s * rz, rz)

    @pl.when(c == 0)
    def _():
        pltpu.sync_copy(uacc.at[rows], u0_out.at[rows])
        pltpu.sync_copy(dacc.at[rows], d0_out.at[rows])

    @pl.when(c == 1)
    def _():
        pltpu.sync_copy(uacc.at[rows], u1_out.at[rows])
        pltpu.sync_copy(dacc.at[rows], d1_out.at[rows])


def _kron_scatter_body(ek_h, dst_h, zu_h, k0_out, k1_out,
                       didx, ekb, kacc, *, n_pad, epw, nchunk):
    c = lax.axis_index("c")
    s = lax.axis_index("s")
    wid = s * _NC + c
    rz = n_pad // _NS
    pltpu.sync_copy(zu_h, kacc.at[pl.ds(s * rz, rz)])
    plsc.subcore_barrier()

    def chunk(i, carry):
        base = wid * epw + i * _C
        pltpu.sync_copy(dst_h.at[pl.ds(base, _C)], didx)
        pltpu.sync_copy(ek_h.at[pl.ds(base, _C)], ekb)
        pltpu.sync_copy(ekb, kacc.at[didx], add=True)
        return carry

    lax.fori_loop(0, nchunk, chunk, 0)
    plsc.subcore_barrier()
    rows = pl.ds(s * rz, rz)

    @pl.when(c == 0)
    def _():
        pltpu.sync_copy(kacc.at[rows], k0_out.at[rows])

    @pl.when(c == 1)
    def _():
        pltpu.sync_copy(kacc.at[rows], k1_out.at[rows])


@jax.jit
def kernel(x, edge_index, Wn, bn, g1, b1, Wk, bk, g2, b2, We, be, Wp, bp,
           Wih, bih, Whh, bhh, g_ln, b_ln, Wc, bc, g3, b3):
    n, d = x.shape
    e = edge_index.shape[1]
    src = edge_index[0]
    dst = edge_index[1]

    wn_pad = jnp.pad(Wn, ((0, 0), (0, 12)))
    bn_pad = jnp.pad(bn, (0, 12)).reshape(1, 32)
    g1_pad = jnp.pad(g1, (0, 12)).reshape(1, 32)
    b1_pad = jnp.pad(b1, (0, 12)).reshape(1, 32)
    wl = jnp.concatenate([We[:d], We[d:]], axis=1)
    bl = jnp.stack([be[0], jnp.zeros((), F32)]).reshape(1, 2)
    wk_pad = jnp.pad(Wk.reshape(20, 20, 128),
                     ((0, 0), (0, 12), (0, 0))).reshape(640, 128)

    nb = 2000
    grid_a = n // nb
    np_pad, hv, l2 = pl.pallas_call(
        _node_pre_body,
        grid=(grid_a,),
        in_specs=[
            pl.BlockSpec((nb, d), lambda i: (i, 0)),
            pl.BlockSpec((d, 32), lambda i: (0, 0)),
            pl.BlockSpec((1, 32), lambda i: (0, 0)),
            pl.BlockSpec((1, 32), lambda i: (0, 0)),
            pl.BlockSpec((1, 32), lambda i: (0, 0)),
            pl.BlockSpec((d, d), lambda i: (0, 0)),
            pl.BlockSpec((1, d), lambda i: (0, 0)),
            pl.BlockSpec((d, 2), lambda i: (0, 0)),
            pl.BlockSpec((1, 2), lambda i: (0, 0)),
        ],
        out_specs=[
            pl.BlockSpec((nb, 32), lambda i: (i, 0)),
            pl.BlockSpec((nb, d), lambda i: (i, 0)),
            pl.BlockSpec((nb, 2), lambda i: (i, 0)),
        ],
        out_shape=[
            jax.ShapeDtypeStruct((n, 32), F32),
            jax.ShapeDtypeStruct((n, d), F32),
            jax.ShapeDtypeStruct((n, 2), F32),
        ],
    )(x, wn_pad, bn_pad, g1_pad, b1_pad, Wp, bp.reshape(1, d), wl, bl)

    ldst = l2[:, 0]
    lsrc = l2[:, 1]

    epw = e // _NW
    nchunk = epw // _C
    n_pad = ((n + 127) // 128) * 128
    rz = n_pad // _NS
    zu = jnp.zeros((rz, d), F32)
    zd = jnp.zeros((rz, 16), F32)
    mesh = plsc.VectorSubcoreMesh(core_axis_name="c", subcore_axis_name="s",
                                  num_cores=_NC, num_subcores=_NS)
    edge_pass = pl.kernel(
        functools.partial(_edge_pass_body, n_pad=n_pad, epw=epw,
                          nchunk=nchunk),
        out_type=[
            jax.ShapeDtypeStruct((n_pad, d), F32),
            jax.ShapeDtypeStruct((n_pad, d), F32),
            jax.ShapeDtypeStruct((n_pad, 16), F32),
            jax.ShapeDtypeStruct((n_pad, 16), F32),
            jax.ShapeDtypeStruct((e, 32), F32),
            jax.ShapeDtypeStruct((e, 32), F32),
        ],
        mesh=mesh,
        compiler_params=pltpu.CompilerParams(needs_layout_passes=False, use_tc_tiling_on_sc=False),
        scratch_types=[
            pltpu.VMEM((_C,), jnp.int32),
            pltpu.VMEM((_C,), jnp.int32),
            pltpu.VMEM((_C, 32), F32),
            pltpu.VMEM((_C, 32), F32),
            pltpu.VMEM((_C, d), F32),
            pltpu.VMEM((_C, 16), F32),
            pltpu.VMEM((n,), F32),
            pltpu.VMEM((n,), F32),
            pltpu.VMEM_SHARED((n_pad, d), F32),
            pltpu.VMEM_SHARED((n_pad, 16), F32),
            pltpu.SemaphoreType.DMA,
            pltpu.SemaphoreType.DMA,
            pltpu.SemaphoreType.DMA,
        ],
    )
    u0, u1, den0, den1, sp, dp = edge_pass(src, dst, np_pad, hv, ldst, lsrc,
                                           zu, zd)
    u0, u1 = u0[:n], u1[:n]
    den0, den1 = den0[:n], den1[:n]

    bsz = 512
    grid_c = e // bsz
    ek = pl.pallas_call(
        _kron_body,
        grid=(grid_c,),
        in_specs=[
            pl.BlockSpec((bsz, 32), lambda i: (i, 0)),
            pl.BlockSpec((bsz, 32), lambda i: (i, 0)),
            pl.BlockSpec((640, d), lambda i: (0, 0)),
            pl.BlockSpec((1, d), lambda i: (0, 0)),
            pl.BlockSpec((1, d), lambda i: (0, 0)),
            pl.BlockSpec((1, d), lambda i: (0, 0)),
        ],
        out_specs=pl.BlockSpec((bsz, d), lambda i: (i, 0)),
        out_shape=jax.ShapeDtypeStruct((e, d), F32),
    )(sp, dp, wk_pad, bk.reshape(1, d), g2.reshape(1, d), b2.reshape(1, d))

    kron_scatter = pl.kernel(
        functools.partial(_kron_scatter_body, n_pad=n_pad, epw=epw,
                          nchunk=nchunk),
        out_type=[
            jax.ShapeDtypeStruct((n_pad, d), F32),
            jax.ShapeDtypeStruct((n_pad, d), F32),
        ],
        mesh=mesh,
        compiler_params=pltpu.CompilerParams(needs_layout_passes=False, use_tc_tiling_on_sc=False),
        scratch_types=[
            pltpu.VMEM((_C,), jnp.int32),
            pltpu.VMEM((_C, d), F32),
            pltpu.VMEM_SHARED((n_pad, d), F32),
        ],
    )
    k0, k1 = kron_scatter(ek, dst, zu)
    k0, k1 = k0[:n], k1[:n]

    out = pl.pallas_call(
        _final_body,
        grid=(grid_a,),
        in_specs=[
            pl.BlockSpec((nb, d), lambda i: (i, 0)),
            pl.BlockSpec((nb, d), lambda i: (i, 0)),
            pl.BlockSpec((nb, d), lambda i: (i, 0)),
            pl.BlockSpec((nb, 16), lambda i: (i, 0)),
            pl.BlockSpec((nb, 16), lambda i: (i, 0)),
            pl.BlockSpec((nb, d), lambda i: (i, 0)),
            pl.BlockSpec((nb, d), lambda i: (i, 0)),
            pl.BlockSpec((d, 3 * d), lambda i: (0, 0)),
            pl.BlockSpec((1, 3 * d), lambda i: (0, 0)),
            pl.BlockSpec((d, 3 * d), lambda i: (0, 0)),
            pl.BlockSpec((1, 3 * d), lambda i: (0, 0)),
            pl.BlockSpec((1, d), lambda i: (0, 0)),
            pl.BlockSpec((1, d), lambda i: (0, 0)),
            pl.BlockSpec((d, d), lambda i: (0, 0)),
            pl.BlockSpec((d, d), lambda i: (0, 0)),
            pl.BlockSpec((1, d), lambda i: (0, 0)),
            pl.BlockSpec((1, d), lambda i: (0, 0)),
            pl.BlockSpec((1, d), lambda i: (0, 0)),
        ],
        out_specs=pl.BlockSpec((nb, d), lambda i: (i, 0)),
        out_shape=jax.ShapeDtypeStruct((n, d), F32),
    )(x, u0, u1, den0, den1, k0, k1,
      Wih.T, bih.reshape(1, 3 * d), Whh.T, bhh.reshape(1, 3 * d),
      g_ln.reshape(1, d), b_ln.reshape(1, d),
      Wc[:d], Wc[d:], bc.reshape(1, d),
      g3.reshape(1, d), b3.reshape(1, d))
    return out
